# baseline (speedup 1.0000x reference)
import functools
import math

import jax
import jax.numpy as jnp
from jax import lax
from jax.experimental import pallas as pl
from jax.experimental.pallas import tpu as pltpu
from jax.experimental.pallas import tpu_sc as plsc

N_NODES = 10000
N_EDGES = 320000
NC = 2
NS = 16
NW = NC * NS
CH = 80
EPW = N_EDGES // NW
NCHUNK = EPW // CH
SCH = 640

BN = 2000
BE = 2560


def _node_body(h_ref, W1_ref, b1_ref, g1_ref, be1_ref, W2_ref, b2_ref, out_ref):
    hh = h_ref[...]
    z = jnp.dot(hh, W1_ref[...], preferred_element_type=jnp.float32) + b1_ref[...]
    mu = jnp.mean(z, axis=1, keepdims=True)
    zc = z - mu
    var = jnp.mean(zc * zc, axis=1, keepdims=True)
    z = zc * lax.rsqrt(var + 1e-5) * g1_ref[...] + be1_ref[...]
    z = jnp.maximum(z, 0.0)
    q = jnp.dot(z, W2_ref[...], preferred_element_type=jnp.float32) + b2_ref[...]
    out_ref[:, :128] = hh
    out_ref[:, 128:] = q


def _node_call(h, W1, b1, g1, be1, W2, b2):
    n = h.shape[0]
    full = lambda i: (0, 0)
    return pl.pallas_call(
        _node_body,
        grid=(n // BN,),
        in_specs=[
            pl.BlockSpec((BN, 128), lambda i: (i, 0)),
            pl.BlockSpec((128, 256), full),
            pl.BlockSpec((1, 256), full),
            pl.BlockSpec((1, 256), full),
            pl.BlockSpec((1, 256), full),
            pl.BlockSpec((256, 128), full),
            pl.BlockSpec((1, 128), full),
        ],
        out_specs=pl.BlockSpec((BN, 256), lambda i: (i, 0)),
        out_shape=jax.ShapeDtypeStruct((n, 256), jnp.float32),
    )(h, W1, b1, g1, be1, W2, b2)


def _gather_call(D, h, dst, src):
    mesh = plsc.VectorSubcoreMesh(core_axis_name="c", subcore_axis_name="s")

    @functools.partial(
        pl.kernel,
        mesh=mesh,
        out_type=[
            jax.ShapeDtypeStruct((N_EDGES, 256), jnp.float32),
            jax.ShapeDtypeStruct((N_EDGES, 128), jnp.float32),
        ],
        scratch_types=[
            pltpu.VMEM((CH,), jnp.int32),
            pltpu.VMEM((CH,), jnp.int32),
            pltpu.VMEM((CH, 256), jnp.float32),
            pltpu.VMEM((CH, 128), jnp.float32),
            pltpu.SemaphoreType.DMA,
            pltpu.SemaphoreType.DMA,
        ],
    )
    def gather_k(D_hbm, h_hbm, dst_hbm, src_hbm, g1_hbm, g2_hbm,
                 dstv, srcv, buf1, buf2, sem1, sem2):
        wid = lax.axis_index("s") * NC + lax.axis_index("c")

        def body(t, carry):
            base = wid * EPW + t * CH
            pltpu.sync_copy(dst_hbm.at[pl.ds(base, CH)], dstv)
            pltpu.sync_copy(src_hbm.at[pl.ds(base, CH)], srcv)
            cp1 = pltpu.async_copy(D_hbm.at[dstv], buf1, sem1)
            cp2 = pltpu.async_copy(h_hbm.at[srcv], buf2, sem2)
            cp1.wait()
            cp2.wait()
            pltpu.sync_copy(buf1, g1_hbm.at[pl.ds(base, CH)])
            pltpu.sync_copy(buf2, g2_hbm.at[pl.ds(base, CH)])
            return carry

        lax.fori_loop(0, NCHUNK, body, 0)

    return gather_k(D, h, dst, src)


def _edge_body(g1_ref, g2_ref, ef_ref, rx_ref, gsel_ref,
               kW1e_ref, kW1d_ref, kW1s_ref, kb1_ref, kg1_ref, kbe1_ref,
               kW2_ref, kb2_ref,
               vW1e_ref, vW1d_ref, vW1s_ref, vb1_ref, vg1_ref, vbe1_ref,
               vW2_ref, vb2_ref,
               p_ref):
    bf = jnp.bfloat16
    hd = g1_ref[:, :128].astype(bf)
    qd = g1_ref[:, 128:]
    hs = g2_ref[...].astype(bf)
    ef = ef_ref[...].astype(bf)

    def front(W1e, W1d, W1s, b1, g1, be1):
        z = (jnp.dot(ef, W1e[...].astype(bf), preferred_element_type=jnp.float32)
             + jnp.dot(hd, W1d[...].astype(bf), preferred_element_type=jnp.float32)
             + jnp.dot(hs, W1s[...].astype(bf), preferred_element_type=jnp.float32)
             + b1[...])
        mu = jnp.mean(z, axis=1, keepdims=True)
        zc = z - mu
        var = jnp.mean(zc * zc, axis=1, keepdims=True)
        z = zc * lax.rsqrt(var + 1e-5) * g1[...] + be1[...]
        return jnp.maximum(z, 0.0).astype(bf)

    zk = front(kW1e_ref, kW1d_ref, kW1s_ref, kb1_ref, kg1_ref, kbe1_ref)
    kk = jnp.dot(zk, kW2_ref[...].astype(bf),
                 preferred_element_type=jnp.float32) + kb2_ref[...]
    zv = front(vW1e_ref, vW1d_ref, vW1s_ref, vb1_ref, vg1_ref, vbe1_ref)
    vv = jnp.dot(zv, vW2_ref[...].astype(bf),
                 preferred_element_type=jnp.float32) + vb2_ref[...]

    t = qd * kk * (1.0 / math.sqrt(8.0))
    logits = jnp.dot(t, gsel_ref[...], preferred_element_type=jnp.float32)
    ex = jnp.exp(logits)
    ev = ex * vv
    rx = rx_ref[...]
    p = jnp.concatenate(
        [ex, ev * rx[:, 0:1], ev * rx[:, 1:2], ev * rx[:, 2:3]], axis=1)
    p_ref[...] = p.T


def _edge_call(G1, G2, ef, rx, gsel, kw, vw):
    e = ef.shape[0]
    full = lambda i: (0, 0)
    wspecs = [
        pl.BlockSpec((16, 256), full),
        pl.BlockSpec((128, 256), full),
        pl.BlockSpec((128, 256), full),
        pl.BlockSpec((1, 256), full),
        pl.BlockSpec((1, 256), full),
        pl.BlockSpec((1, 256), full),
    ]
    return pl.pallas_call(
        _edge_body,
        grid=(e // BE,),
        in_specs=[
            pl.BlockSpec((BE, 256), lambda i: (i, 0)),
            pl.BlockSpec((BE, 128), lambda i: (i, 0)),
            pl.BlockSpec((BE, 16), lambda i: (i, 0)),
            pl.BlockSpec((BE, 3), lambda i: (i, 0)),
            pl.BlockSpec((128, 16), full),
        ] + wspecs + [
            pl.BlockSpec((256, 128), full),
            pl.BlockSpec((1, 128), full),
        ] + wspecs + [
            pl.BlockSpec((256, 16), full),
            pl.BlockSpec((1, 16), full),
        ],
        out_specs=pl.BlockSpec((64, BE), lambda i: (0, i)),
        out_shape=jax.ShapeDtypeStruct((64, e), jnp.float32),
    )(G1, G2, ef, rx, gsel, *kw, *vw)


def _scatter_call(PT, dst):
    mesh = plsc.VectorSubcoreMesh(core_axis_name="c", subcore_axis_name="s")
    eph = N_EDGES // NC // 2
    nchunk = eph // SCH

    @functools.partial(
        pl.kernel,
        mesh=mesh,
        out_type=jax.ShapeDtypeStruct((NC, 2, 64, N_NODES), jnp.float32),
        compiler_params=pltpu.CompilerParams(needs_layout_passes=False),
        scratch_types=[
            pltpu.VMEM((SCH,), jnp.int32),
            pltpu.VMEM((8, SCH), jnp.float32),
        ] + [pltpu.VMEM((N_NODES,), jnp.float32) for _ in range(8)],
    )
    def scatter_k(PT_hbm, dst_hbm, out_hbm, dstv, pbuf, *accs):
        cid = lax.axis_index("c")
        sid = lax.axis_index("s")
        g = sid % 8
        hh = sid // 8

        zero16 = jnp.zeros((16,), jnp.float32)

        def zrow(r, carry):
            for c in range(8):
                accs[c][pl.ds(r * 16, 16)] = zero16
            return carry

        lax.fori_loop(0, N_NODES // 16, zrow, 0)

        def body(t, carry):
            base = (cid * 2 + hh) * eph + t * SCH
            pltpu.sync_copy(dst_hbm.at[pl.ds(base, SCH)], dstv)
            pltpu.sync_copy(PT_hbm.at[pl.ds(8 * g, 8), pl.ds(base, SCH)], pbuf)

            def group(j, carry2):
                rows = dstv[pl.ds(j * 16, 16)]
                for c in range(8):
                    vals = pbuf[c, pl.ds(j * 16, 16)]
                    plsc.addupdate_scatter(accs[c], [rows], vals)
                return carry2

            lax.fori_loop(0, SCH // 16, group, 0)
            return carry

        lax.fori_loop(0, nchunk, body, 0)
        for c in range(8):
            pltpu.sync_copy(accs[c], out_hbm.at[cid, hh, 8 * g + c])

    return scatter_k(PT, dst)


def _fin_body(a_ref, b_ref, c_ref, d_ref, o_ref):
    acc = a_ref[...] + b_ref[...] + c_ref[...] + d_ref[...]
    s = acc[:16, :] + 1e-16
    rows = [jnp.sum(acc[16 + 16 * c: 32 + 16 * c, :] / s, axis=0, keepdims=True)
            * (1.0 / 16.0) for c in range(3)]
    o_ref[...] = jnp.concatenate(rows, axis=0)


def _fin_call(parts):
    n = parts[0].shape[1]
    return pl.pallas_call(
        _fin_body,
        grid=(1,),
        in_specs=[pl.BlockSpec((64, n), lambda i: (0, 0)) for _ in range(4)],
        out_specs=pl.BlockSpec((3, n), lambda i: (0, 0)),
        out_shape=jax.ShapeDtypeStruct((3, n), jnp.float32),
    )(*parts)


def kernel(h, rel_x, edge_feat, edge_index,
           k_W1, k_b1, k_g1, k_be1, k_W2, k_b2,
           v_W1, v_b1, v_g1, v_be1, v_W2, v_b2,
           q_W1, q_b1, q_g1, q_be1, q_W2, q_b2):
    src = edge_index[0].astype(jnp.int32)
    dst = edge_index[1].astype(jnp.int32)
    row = lambda x: x.reshape(1, -1)

    D = _node_call(h, q_W1, row(q_b1), row(q_g1), row(q_be1), q_W2, row(q_b2))
    G1, G2 = _gather_call(D, h, dst, src)

    gsel = jnp.kron(jnp.eye(16, dtype=jnp.float32),
                    jnp.ones((8, 1), dtype=jnp.float32))
    kw = (k_W1[:16], k_W1[16:144], k_W1[144:], row(k_b1), row(k_g1),
          row(k_be1), k_W2, row(k_b2))
    vw = (v_W1[:16], v_W1[16:144], v_W1[144:], row(v_b1), row(v_g1),
          row(v_be1), v_W2, row(v_b2))
    P = _edge_call(G1, G2, edge_feat, rel_x, gsel, kw, vw)

    partials = _scatter_call(P, dst)
    parts = [partials[0, 0], partials[0, 1], partials[1, 0], partials[1, 1]]
    return _fin_call(parts).T


# speedup vs baseline: 1.2123x; 1.2123x over previous
import functools
import math

import jax
import jax.numpy as jnp
from jax import lax
from jax.experimental import pallas as pl
from jax.experimental.pallas import tpu as pltpu
from jax.experimental.pallas import tpu_sc as plsc

N_NODES = 10000
N_EDGES = 320000
NC = 2
NS = 16
NW = NC * NS
CH = 80
EPW = N_EDGES // NW
NCHUNK = EPW // CH
SCH = 640

BN = 2000
BE = 2560


def _node_body(h_ref, W1_ref, b1_ref, g1_ref, be1_ref, W2_ref, b2_ref, out_ref):
    hh = h_ref[...]
    z = jnp.dot(hh, W1_ref[...], preferred_element_type=jnp.float32) + b1_ref[...]
    mu = jnp.mean(z, axis=1, keepdims=True)
    zc = z - mu
    var = jnp.mean(zc * zc, axis=1, keepdims=True)
    z = zc * lax.rsqrt(var + 1e-5) * g1_ref[...] + be1_ref[...]
    z = jnp.maximum(z, 0.0)
    q = jnp.dot(z, W2_ref[...], preferred_element_type=jnp.float32) + b2_ref[...]
    out_ref[:, :128] = hh
    out_ref[:, 128:] = q


def _node_call(h, W1, b1, g1, be1, W2, b2):
    n = h.shape[0]
    full = lambda i: (0, 0)
    return pl.pallas_call(
        _node_body,
        grid=(n // BN,),
        in_specs=[
            pl.BlockSpec((BN, 128), lambda i: (i, 0)),
            pl.BlockSpec((128, 256), full),
            pl.BlockSpec((1, 256), full),
            pl.BlockSpec((1, 256), full),
            pl.BlockSpec((1, 256), full),
            pl.BlockSpec((256, 128), full),
            pl.BlockSpec((1, 128), full),
        ],
        out_specs=pl.BlockSpec((BN, 256), lambda i: (i, 0)),
        out_shape=jax.ShapeDtypeStruct((n, 256), jnp.float32),
    )(h, W1, b1, g1, be1, W2, b2)


def _gather_call(D, h, dst, src):
    mesh = plsc.VectorSubcoreMesh(core_axis_name="c", subcore_axis_name="s")

    @functools.partial(
        pl.kernel,
        mesh=mesh,
        out_type=[
            jax.ShapeDtypeStruct((N_EDGES, 256), jnp.float32),
            jax.ShapeDtypeStruct((N_EDGES, 128), jnp.float32),
        ],
        scratch_types=[
            pltpu.VMEM((CH,), jnp.int32),
            pltpu.VMEM((CH,), jnp.int32),
            pltpu.VMEM((CH, 256), jnp.float32),
            pltpu.VMEM((CH, 128), jnp.float32),
            pltpu.SemaphoreType.DMA,
            pltpu.SemaphoreType.DMA,
        ],
    )
    def gather_k(D_hbm, h_hbm, dst_hbm, src_hbm, g1_hbm, g2_hbm,
                 dstv, srcv, buf1, buf2, sem1, sem2):
        wid = lax.axis_index("s") * NC + lax.axis_index("c")

        def body(t, carry):
            base = wid * EPW + t * CH
            pltpu.sync_copy(dst_hbm.at[pl.ds(base, CH)], dstv)
            pltpu.sync_copy(src_hbm.at[pl.ds(base, CH)], srcv)
            cp1 = pltpu.async_copy(D_hbm.at[dstv], buf1, sem1)
            cp2 = pltpu.async_copy(h_hbm.at[srcv], buf2, sem2)
            cp1.wait()
            cp2.wait()
            pltpu.sync_copy(buf1, g1_hbm.at[pl.ds(base, CH)])
            pltpu.sync_copy(buf2, g2_hbm.at[pl.ds(base, CH)])
            return carry

        lax.fori_loop(0, NCHUNK, body, 0)

    return gather_k(D, h, dst, src)


def _edge_body(g1_ref, g2_ref, ef_ref, rx_ref, gsel_ref,
               W1_ref, b1_ref, g1w_ref, be1_ref, W2_ref, b2_ref,
               p_ref):
    bf = jnp.bfloat16
    hd = g1_ref[:, :128].astype(bf)
    qd = g1_ref[:, 128:]
    hs = g2_ref[...].astype(bf)
    ef = ef_ref[...].astype(bf)

    x = jnp.concatenate([ef, hd, hs], axis=1)
    z = jnp.dot(x, W1_ref[...], preferred_element_type=jnp.float32) + b1_ref[...]

    def norm(zz, g1w, be1):
        mu = jnp.mean(zz, axis=1, keepdims=True)
        zc = zz - mu
        var = jnp.mean(zc * zc, axis=1, keepdims=True)
        zz = zc * lax.rsqrt(var + 1e-5) * g1w + be1
        return jnp.maximum(zz, 0.0).astype(bf)

    rk = norm(z[:, :256], g1w_ref[:, :256], be1_ref[:, :256])
    rv = norm(z[:, 256:], g1w_ref[:, 256:], be1_ref[:, 256:])
    r = jnp.concatenate([rk, rv], axis=1)
    kv = jnp.dot(r, W2_ref[...], preferred_element_type=jnp.float32) + b2_ref[...]
    kk = kv[:, :128]
    vv = kv[:, 128:144]

    t = (qd * kk * (1.0 / math.sqrt(8.0))).astype(bf)
    logits = jnp.dot(t, gsel_ref[...], preferred_element_type=jnp.float32)
    ex = jnp.exp(logits)
    ev = ex * vv
    ext = ex.T
    evt = ev.T
    rxt = rx_ref[...].T
    p_ref[...] = jnp.concatenate(
        [ext, evt * rxt[0:1], evt * rxt[1:2], evt * rxt[2:3]], axis=0)


def _edge_call(G1, G2, ef, rx, gsel, W1, b1, g1w, be1, W2, b2):
    e = ef.shape[0]
    full = lambda i: (0, 0)
    return pl.pallas_call(
        _edge_body,
        grid=(e // BE,),
        in_specs=[
            pl.BlockSpec((BE, 256), lambda i: (i, 0)),
            pl.BlockSpec((BE, 128), lambda i: (i, 0)),
            pl.BlockSpec((BE, 16), lambda i: (i, 0)),
            pl.BlockSpec((BE, 3), lambda i: (i, 0)),
            pl.BlockSpec((128, 16), full),
            pl.BlockSpec((272, 512), full),
            pl.BlockSpec((1, 512), full),
            pl.BlockSpec((1, 512), full),
            pl.BlockSpec((1, 512), full),
            pl.BlockSpec((512, 144), full),
            pl.BlockSpec((1, 144), full),
        ],
        out_specs=pl.BlockSpec((64, BE), lambda i: (0, i)),
        out_shape=jax.ShapeDtypeStruct((64, e), jnp.float32),
    )(G1, G2, ef, rx, gsel, W1, b1, g1w, be1, W2, b2)


def _scatter_call(PT, dst):
    mesh = plsc.VectorSubcoreMesh(core_axis_name="c", subcore_axis_name="s")
    eph = N_EDGES // NC // 2
    nchunk = eph // SCH

    @functools.partial(
        pl.kernel,
        mesh=mesh,
        out_type=jax.ShapeDtypeStruct((NC, 2, 64, N_NODES), jnp.float32),
        compiler_params=pltpu.CompilerParams(needs_layout_passes=False),
        scratch_types=[
            pltpu.VMEM((SCH,), jnp.int32),
            pltpu.VMEM((8, SCH), jnp.float32),
        ] + [pltpu.VMEM((N_NODES,), jnp.float32) for _ in range(8)],
    )
    def scatter_k(PT_hbm, dst_hbm, out_hbm, dstv, pbuf, *accs):
        cid = lax.axis_index("c")
        sid = lax.axis_index("s")
        g = sid % 8
        hh = sid // 8

        zero16 = jnp.zeros((16,), jnp.float32)

        def zrow(r, carry):
            for c in range(8):
                accs[c][pl.ds(r * 16, 16)] = zero16
            return carry

        lax.fori_loop(0, N_NODES // 16, zrow, 0)

        def body(t, carry):
            base = (cid * 2 + hh) * eph + t * SCH
            pltpu.sync_copy(dst_hbm.at[pl.ds(base, SCH)], dstv)
            pltpu.sync_copy(PT_hbm.at[pl.ds(8 * g, 8), pl.ds(base, SCH)], pbuf)

            def group(j, carry2):
                rows = dstv[pl.ds(j * 16, 16)]
                for c in range(8):
                    vals = pbuf[c, pl.ds(j * 16, 16)]
                    plsc.addupdate_scatter(accs[c], [rows], vals)
                return carry2

            lax.fori_loop(0, SCH // 16, group, 0)
            return carry

        lax.fori_loop(0, nchunk, body, 0)
        for c in range(8):
            pltpu.sync_copy(accs[c], out_hbm.at[cid, hh, 8 * g + c])

    return scatter_k(PT, dst)


def _fin_body(a_ref, b_ref, c_ref, d_ref, o_ref):
    acc = a_ref[...] + b_ref[...] + c_ref[...] + d_ref[...]
    s = acc[:16, :] + 1e-16
    rows = [jnp.sum(acc[16 + 16 * c: 32 + 16 * c, :] / s, axis=0, keepdims=True)
            * (1.0 / 16.0) for c in range(3)]
    o_ref[...] = jnp.concatenate(rows, axis=0)


def _fin_call(parts):
    n = parts[0].shape[1]
    return pl.pallas_call(
        _fin_body,
        grid=(1,),
        in_specs=[pl.BlockSpec((64, n), lambda i: (0, 0)) for _ in range(4)],
        out_specs=pl.BlockSpec((3, n), lambda i: (0, 0)),
        out_shape=jax.ShapeDtypeStruct((3, n), jnp.float32),
    )(*parts)


def kernel(h, rel_x, edge_feat, edge_index,
           k_W1, k_b1, k_g1, k_be1, k_W2, k_b2,
           v_W1, v_b1, v_g1, v_be1, v_W2, v_b2,
           q_W1, q_b1, q_g1, q_be1, q_W2, q_b2):
    src = edge_index[0].astype(jnp.int32)
    dst = edge_index[1].astype(jnp.int32)
    row = lambda x: x.reshape(1, -1)

    D = _node_call(h, q_W1, row(q_b1), row(q_g1), row(q_be1), q_W2, row(q_b2))
    G1, G2 = _gather_call(D, h, dst, src)

    bf = jnp.bfloat16
    gsel = jnp.kron(jnp.eye(16, dtype=bf), jnp.ones((8, 1), dtype=bf))
    W1 = jnp.concatenate([k_W1, v_W1], axis=1).astype(bf)
    b1 = row(jnp.concatenate([k_b1, v_b1]))
    g1w = row(jnp.concatenate([k_g1, v_g1]))
    be1 = row(jnp.concatenate([k_be1, v_be1]))
    W2 = jnp.block([
        [k_W2, jnp.zeros((256, 16), jnp.float32)],
        [jnp.zeros((256, 128), jnp.float32), v_W2],
    ]).astype(bf)
    b2 = row(jnp.concatenate([k_b2, v_b2]))
    P = _edge_call(G1, G2, edge_feat, rel_x, gsel, W1, b1, g1w, be1, W2, b2)

    partials = _scatter_call(P, dst)
    parts = [partials[0, 0], partials[0, 1], partials[1, 0], partials[1, 1]]
    return _fin_call(parts).T


# speedup vs baseline: 1.2992x; 1.0717x over previous
import functools
import math

import jax
import jax.numpy as jnp
from jax import lax
from jax.experimental import pallas as pl
from jax.experimental.pallas import tpu as pltpu
from jax.experimental.pallas import tpu_sc as plsc

N_NODES = 10000
N_EDGES = 320000
NC = 2
NS = 16
NW = NC * NS
CH = 80
EPW = N_EDGES // NW
NCHUNK = EPW // CH
SCH = 640

BN = 2000
BE = 2560


def _node_body(h_ref, W1_ref, b1_ref, g1_ref, be1_ref, W2_ref, b2_ref, out_ref):
    hh = h_ref[...]
    z = jnp.dot(hh, W1_ref[...], preferred_element_type=jnp.float32) + b1_ref[...]
    mu = jnp.mean(z, axis=1, keepdims=True)
    zc = z - mu
    var = jnp.mean(zc * zc, axis=1, keepdims=True)
    z = zc * lax.rsqrt(var + 1e-5) * g1_ref[...] + be1_ref[...]
    z = jnp.maximum(z, 0.0)
    q = jnp.dot(z, W2_ref[...], preferred_element_type=jnp.float32) + b2_ref[...]
    out_ref[:, :128] = hh
    out_ref[:, 128:] = q


def _node_call(h, W1, b1, g1, be1, W2, b2):
    n = h.shape[0]
    full = lambda i: (0, 0)
    return pl.pallas_call(
        _node_body,
        grid=(n // BN,),
        in_specs=[
            pl.BlockSpec((BN, 128), lambda i: (i, 0)),
            pl.BlockSpec((128, 256), full),
            pl.BlockSpec((1, 256), full),
            pl.BlockSpec((1, 256), full),
            pl.BlockSpec((1, 256), full),
            pl.BlockSpec((256, 128), full),
            pl.BlockSpec((1, 128), full),
        ],
        out_specs=pl.BlockSpec((BN, 256), lambda i: (i, 0)),
        out_shape=jax.ShapeDtypeStruct((n, 256), jnp.float32),
    )(h, W1, b1, g1, be1, W2, b2)


def _gather_call(D, h, dst, src):
    mesh = plsc.VectorSubcoreMesh(core_axis_name="c", subcore_axis_name="s")

    @functools.partial(
        pl.kernel,
        mesh=mesh,
        out_type=[
            jax.ShapeDtypeStruct((N_EDGES, 256), jnp.float32),
            jax.ShapeDtypeStruct((N_EDGES, 128), jnp.float32),
        ],
        scratch_types=[
            pltpu.VMEM((EPW,), jnp.int32),
            pltpu.VMEM((EPW,), jnp.int32),
            pltpu.VMEM((2, CH, 256), jnp.float32),
            pltpu.VMEM((2, CH, 128), jnp.float32),
        ] + [pltpu.SemaphoreType.DMA] * 8,
    )
    def gather_k(D_hbm, h_hbm, dst_hbm, src_hbm, g1_hbm, g2_hbm,
                 dsti, srci, buf1, buf2, *sems):
        wid = lax.axis_index("s") * NC + lax.axis_index("c")
        base0 = wid * EPW
        pltpu.sync_copy(dst_hbm.at[pl.ds(base0, EPW)], dsti)
        pltpu.sync_copy(src_hbm.at[pl.ds(base0, EPW)], srci)
        g1s, g2s, w1s, w2s = sems[0:2], sems[2:4], sems[4:6], sems[6:8]

        def start_g(t, s):
            pltpu.async_copy(D_hbm.at[dsti.at[pl.ds(t * CH, CH)]],
                             buf1.at[s], g1s[s])
            pltpu.async_copy(h_hbm.at[srci.at[pl.ds(t * CH, CH)]],
                             buf2.at[s], g2s[s])

        def wait_g(s):
            pltpu.make_async_copy(D_hbm.at[dsti.at[pl.ds(0, CH)]],
                                  buf1.at[s], g1s[s]).wait()
            pltpu.make_async_copy(h_hbm.at[srci.at[pl.ds(0, CH)]],
                                  buf2.at[s], g2s[s]).wait()

        def start_w(t, s):
            base = base0 + t * CH
            pltpu.async_copy(buf1.at[s], g1_hbm.at[pl.ds(base, CH)], w1s[s])
            pltpu.async_copy(buf2.at[s], g2_hbm.at[pl.ds(base, CH)], w2s[s])

        def wait_w(s):
            pltpu.make_async_copy(buf1.at[s], g1_hbm.at[pl.ds(0, CH)],
                                  w1s[s]).wait()
            pltpu.make_async_copy(buf2.at[s], g2_hbm.at[pl.ds(0, CH)],
                                  w2s[s]).wait()

        start_g(0, 0)

        def body(u, carry):
            t0 = 2 * u

            @pl.when(u >= 1)
            def _():
                wait_w(1)

            @pl.when(t0 + 1 < NCHUNK)
            def _():
                start_g(t0 + 1, 1)

            wait_g(0)
            start_w(t0, 0)

            @pl.when(t0 + 1 < NCHUNK)
            def _():
                wait_w(0)

                @pl.when(t0 + 2 < NCHUNK)
                def _():
                    start_g(t0 + 2, 0)

                wait_g(1)
                start_w(t0 + 1, 1)

            return carry

        lax.fori_loop(0, (NCHUNK + 1) // 2, body, 0)
        wait_w((NCHUNK - 1) % 2)

    return gather_k(D, h, dst, src)


def _edge_body(g1_ref, g2_ref, ef_ref, rx_ref, gsel_ref,
               W1_ref, b1_ref, g1w_ref, be1_ref, W2_ref, b2_ref,
               p_ref):
    bf = jnp.bfloat16
    hd = g1_ref[:, :128].astype(bf)
    qd = g1_ref[:, 128:]
    hs = g2_ref[...].astype(bf)
    ef = ef_ref[...].astype(bf)

    x = jnp.concatenate([ef, hd, hs], axis=1)
    z = jnp.dot(x, W1_ref[...], preferred_element_type=jnp.float32) + b1_ref[...]

    def norm(zz, g1w, be1):
        mu = jnp.mean(zz, axis=1, keepdims=True)
        zc = zz - mu
        var = jnp.mean(zc * zc, axis=1, keepdims=True)
        zz = zc * lax.rsqrt(var + 1e-5) * g1w + be1
        return jnp.maximum(zz, 0.0).astype(bf)

    rk = norm(z[:, :256], g1w_ref[:, :256], be1_ref[:, :256])
    rv = norm(z[:, 256:], g1w_ref[:, 256:], be1_ref[:, 256:])
    r = jnp.concatenate([rk, rv], axis=1)
    kv = jnp.dot(r, W2_ref[...], preferred_element_type=jnp.float32) + b2_ref[...]
    kk = kv[:, :128]
    vv = kv[:, 128:144]

    t = (qd * kk * (1.0 / math.sqrt(8.0))).astype(bf)
    logits = jnp.dot(t, gsel_ref[...], preferred_element_type=jnp.float32)
    ex = jnp.exp(logits)
    ev = ex * vv
    ext = ex.T
    evt = ev.T
    rxt = rx_ref[...].T
    p_ref[...] = jnp.concatenate(
        [ext, evt * rxt[0:1], evt * rxt[1:2], evt * rxt[2:3]], axis=0)


def _edge_call(G1, G2, ef, rx, gsel, W1, b1, g1w, be1, W2, b2):
    e = ef.shape[0]
    full = lambda i: (0, 0)
    return pl.pallas_call(
        _edge_body,
        grid=(e // BE,),
        in_specs=[
            pl.BlockSpec((BE, 256), lambda i: (i, 0)),
            pl.BlockSpec((BE, 128), lambda i: (i, 0)),
            pl.BlockSpec((BE, 16), lambda i: (i, 0)),
            pl.BlockSpec((BE, 3), lambda i: (i, 0)),
            pl.BlockSpec((128, 16), full),
            pl.BlockSpec((272, 512), full),
            pl.BlockSpec((1, 512), full),
            pl.BlockSpec((1, 512), full),
            pl.BlockSpec((1, 512), full),
            pl.BlockSpec((512, 144), full),
            pl.BlockSpec((1, 144), full),
        ],
        out_specs=pl.BlockSpec((64, BE), lambda i: (0, i)),
        out_shape=jax.ShapeDtypeStruct((64, e), jnp.float32),
    )(G1, G2, ef, rx, gsel, W1, b1, g1w, be1, W2, b2)


def _scatter_call(PT, dst):
    mesh = plsc.VectorSubcoreMesh(core_axis_name="c", subcore_axis_name="s")
    eph = N_EDGES // NC // 2
    nchunk = eph // SCH

    @functools.partial(
        pl.kernel,
        mesh=mesh,
        out_type=jax.ShapeDtypeStruct((NC, 2, 64, N_NODES), jnp.float32),
        compiler_params=pltpu.CompilerParams(needs_layout_passes=False),
        scratch_types=[
            pltpu.VMEM((SCH,), jnp.int32),
            pltpu.VMEM((8, SCH), jnp.float32),
        ] + [pltpu.VMEM((N_NODES,), jnp.float32) for _ in range(8)],
    )
    def scatter_k(PT_hbm, dst_hbm, out_hbm, dstv, pbuf, *accs):
        cid = lax.axis_index("c")
        sid = lax.axis_index("s")
        g = sid % 8
        hh = sid // 8

        zero16 = jnp.zeros((16,), jnp.float32)

        def zrow(r, carry):
            for c in range(8):
                accs[c][pl.ds(r * 16, 16)] = zero16
            return carry

        lax.fori_loop(0, N_NODES // 16, zrow, 0)

        def body(t, carry):
            base = (cid * 2 + hh) * eph + t * SCH
            pltpu.sync_copy(dst_hbm.at[pl.ds(base, SCH)], dstv)
            pltpu.sync_copy(PT_hbm.at[pl.ds(8 * g, 8), pl.ds(base, SCH)], pbuf)

            def group(j, carry2):
                rows = dstv[pl.ds(j * 16, 16)]
                for c in range(8):
                    vals = pbuf[c, pl.ds(j * 16, 16)]
                    plsc.addupdate_scatter(accs[c], [rows], vals)
                return carry2

            lax.fori_loop(0, SCH // 16, group, 0)
            return carry

        lax.fori_loop(0, nchunk, body, 0)
        for c in range(8):
            pltpu.sync_copy(accs[c], out_hbm.at[cid, hh, 8 * g + c])

    return scatter_k(PT, dst)


def _fin_body(a_ref, b_ref, c_ref, d_ref, o_ref):
    acc = a_ref[...] + b_ref[...] + c_ref[...] + d_ref[...]
    s = acc[:16, :] + 1e-16
    rows = [jnp.sum(acc[16 + 16 * c: 32 + 16 * c, :] / s, axis=0, keepdims=True)
            * (1.0 / 16.0) for c in range(3)]
    o_ref[...] = jnp.concatenate(rows, axis=0)


def _fin_call(parts):
    n = parts[0].shape[1]
    return pl.pallas_call(
        _fin_body,
        grid=(1,),
        in_specs=[pl.BlockSpec((64, n), lambda i: (0, 0)) for _ in range(4)],
        out_specs=pl.BlockSpec((3, n), lambda i: (0, 0)),
        out_shape=jax.ShapeDtypeStruct((3, n), jnp.float32),
    )(*parts)


def kernel(h, rel_x, edge_feat, edge_index,
           k_W1, k_b1, k_g1, k_be1, k_W2, k_b2,
           v_W1, v_b1, v_g1, v_be1, v_W2, v_b2,
           q_W1, q_b1, q_g1, q_be1, q_W2, q_b2):
    src = edge_index[0].astype(jnp.int32)
    dst = edge_index[1].astype(jnp.int32)
    row = lambda x: x.reshape(1, -1)

    D = _node_call(h, q_W1, row(q_b1), row(q_g1), row(q_be1), q_W2, row(q_b2))
    G1, G2 = _gather_call(D, h, dst, src)

    bf = jnp.bfloat16
    gsel = jnp.kron(jnp.eye(16, dtype=bf), jnp.ones((8, 1), dtype=bf))
    W1 = jnp.concatenate([k_W1, v_W1], axis=1).astype(bf)
    b1 = row(jnp.concatenate([k_b1, v_b1]))
    g1w = row(jnp.concatenate([k_g1, v_g1]))
    be1 = row(jnp.concatenate([k_be1, v_be1]))
    W2 = jnp.block([
        [k_W2, jnp.zeros((256, 16), jnp.float32)],
        [jnp.zeros((256, 128), jnp.float32), v_W2],
    ]).astype(bf)
    b2 = row(jnp.concatenate([k_b2, v_b2]))
    P = _edge_call(G1, G2, edge_feat, rel_x, gsel, W1, b1, g1w, be1, W2, b2)

    partials = _scatter_call(P, dst)
    parts = [partials[0, 0], partials[0, 1], partials[1, 0], partials[1, 1]]
    return _fin_call(parts).T


# speedup vs baseline: 1.4710x; 1.1323x over previous
import functools
import math

import jax
import jax.numpy as jnp
from jax import lax
from jax.experimental import pallas as pl
from jax.experimental.pallas import tpu as pltpu
from jax.experimental.pallas import tpu_sc as plsc

N_NODES = 10000
N_EDGES = 320000
NC = 2
NS = 16
NW = NC * NS
CH = 80
EPW = N_EDGES // NW
NCHUNK = EPW // CH
SCH = 640

BN = 2000
BE = 2560


def _node_body(h_ref, W1_ref, b1_ref, g1_ref, be1_ref, W2_ref, b2_ref, out_ref):
    hh = h_ref[...]
    z = jnp.dot(hh, W1_ref[...], preferred_element_type=jnp.float32) + b1_ref[...]
    mu = jnp.mean(z, axis=1, keepdims=True)
    zc = z - mu
    var = jnp.mean(zc * zc, axis=1, keepdims=True)
    z = zc * lax.rsqrt(var + 1e-5) * g1_ref[...] + be1_ref[...]
    z = jnp.maximum(z, 0.0)
    q = jnp.dot(z, W2_ref[...], preferred_element_type=jnp.float32) + b2_ref[...]
    out_ref[:, :128] = hh
    out_ref[:, 128:] = q


def _node_call(h, W1, b1, g1, be1, W2, b2):
    n = h.shape[0]
    full = lambda i: (0, 0)
    return pl.pallas_call(
        _node_body,
        grid=(n // BN,),
        in_specs=[
            pl.BlockSpec((BN, 128), lambda i: (i, 0)),
            pl.BlockSpec((128, 256), full),
            pl.BlockSpec((1, 256), full),
            pl.BlockSpec((1, 256), full),
            pl.BlockSpec((1, 256), full),
            pl.BlockSpec((256, 128), full),
            pl.BlockSpec((1, 128), full),
        ],
        out_specs=pl.BlockSpec((BN, 256), lambda i: (i, 0)),
        out_shape=jax.ShapeDtypeStruct((n, 256), jnp.float32),
    )(h, W1, b1, g1, be1, W2, b2)


def _gather_call(D, h, dst, src):
    mesh = plsc.VectorSubcoreMesh(core_axis_name="c", subcore_axis_name="s")

    @functools.partial(
        pl.kernel,
        mesh=mesh,
        out_type=[
            jax.ShapeDtypeStruct((N_EDGES, 256), jnp.float32),
            jax.ShapeDtypeStruct((N_EDGES, 128), jnp.float32),
        ],
        scratch_types=[
            pltpu.VMEM((EPW,), jnp.int32),
            pltpu.VMEM((EPW,), jnp.int32),
            pltpu.VMEM((2, CH, 256), jnp.float32),
            pltpu.VMEM((2, CH, 128), jnp.float32),
        ] + [pltpu.SemaphoreType.DMA] * 8,
    )
    def gather_k(D_hbm, h_hbm, dst_hbm, src_hbm, g1_hbm, g2_hbm,
                 dsti, srci, buf1, buf2, *sems):
        wid = lax.axis_index("s") * NC + lax.axis_index("c")
        base0 = wid * EPW
        pltpu.sync_copy(dst_hbm.at[pl.ds(base0, EPW)], dsti)
        pltpu.sync_copy(src_hbm.at[pl.ds(base0, EPW)], srci)
        g1s, g2s, w1s, w2s = sems[0:2], sems[2:4], sems[4:6], sems[6:8]

        def start_g(t, s):
            pltpu.async_copy(D_hbm.at[dsti.at[pl.ds(t * CH, CH)]],
                             buf1.at[s], g1s[s])
            pltpu.async_copy(h_hbm.at[srci.at[pl.ds(t * CH, CH)]],
                             buf2.at[s], g2s[s])

        def wait_g(s):
            pltpu.make_async_copy(D_hbm.at[dsti.at[pl.ds(0, CH)]],
                                  buf1.at[s], g1s[s]).wait()
            pltpu.make_async_copy(h_hbm.at[srci.at[pl.ds(0, CH)]],
                                  buf2.at[s], g2s[s]).wait()

        def start_w(t, s):
            base = base0 + t * CH
            pltpu.async_copy(buf1.at[s], g1_hbm.at[pl.ds(base, CH)], w1s[s])
            pltpu.async_copy(buf2.at[s], g2_hbm.at[pl.ds(base, CH)], w2s[s])

        def wait_w(s):
            pltpu.make_async_copy(buf1.at[s], g1_hbm.at[pl.ds(0, CH)],
                                  w1s[s]).wait()
            pltpu.make_async_copy(buf2.at[s], g2_hbm.at[pl.ds(0, CH)],
                                  w2s[s]).wait()

        start_g(0, 0)

        def body(u, carry):
            t0 = 2 * u

            @pl.when(u >= 1)
            def _():
                wait_w(1)

            @pl.when(t0 + 1 < NCHUNK)
            def _():
                start_g(t0 + 1, 1)

            wait_g(0)
            start_w(t0, 0)

            @pl.when(t0 + 1 < NCHUNK)
            def _():
                wait_w(0)

                @pl.when(t0 + 2 < NCHUNK)
                def _():
                    start_g(t0 + 2, 0)

                wait_g(1)
                start_w(t0 + 1, 1)

            return carry

        lax.fori_loop(0, (NCHUNK + 1) // 2, body, 0)
        wait_w((NCHUNK - 1) % 2)

    return gather_k(D, h, dst, src)


def _edge_body(g1_ref, g2_ref, ef_ref, rx_ref, gsel_ref,
               W1_ref, b1_ref, g1w_ref, be1_ref, W2_ref, b2_ref,
               p_ref):
    bf = jnp.bfloat16
    hd = g1_ref[:, :128].astype(bf)
    qd = g1_ref[:, 128:]
    hs = g2_ref[...].astype(bf)
    ef = ef_ref[...].astype(bf)

    x = jnp.concatenate([ef, hd, hs], axis=1)
    z = jnp.dot(x, W1_ref[...], preferred_element_type=jnp.float32) + b1_ref[...]

    def norm(zz, g1w, be1):
        mu = jnp.mean(zz, axis=1, keepdims=True)
        zc = zz - mu
        var = jnp.mean(zc * zc, axis=1, keepdims=True)
        zz = zc * lax.rsqrt(var + 1e-5) * g1w + be1
        return jnp.maximum(zz, 0.0).astype(bf)

    rk = norm(z[:, :256], g1w_ref[:, :256], be1_ref[:, :256])
    rv = norm(z[:, 256:], g1w_ref[:, 256:], be1_ref[:, 256:])
    r = jnp.concatenate([rk, rv], axis=1)
    kv = jnp.dot(r, W2_ref[...], preferred_element_type=jnp.float32) + b2_ref[...]
    kk = kv[:, :128]
    vv = kv[:, 128:144]

    t = (qd * kk * (1.0 / math.sqrt(8.0))).astype(bf)
    logits = jnp.dot(t, gsel_ref[...], preferred_element_type=jnp.float32)
    ex = jnp.exp(logits)
    ev = ex * vv
    ext = ex.T
    evt = ev.T
    rxt = rx_ref[...].T
    p_ref[...] = jnp.concatenate(
        [ext, evt * rxt[0:1], evt * rxt[1:2], evt * rxt[2:3]], axis=0)


def _edge_call(G1, G2, ef, rx, gsel, W1, b1, g1w, be1, W2, b2):
    e = ef.shape[0]
    full = lambda i: (0, 0)
    return pl.pallas_call(
        _edge_body,
        grid=(e // BE,),
        in_specs=[
            pl.BlockSpec((BE, 256), lambda i: (i, 0)),
            pl.BlockSpec((BE, 128), lambda i: (i, 0)),
            pl.BlockSpec((BE, 16), lambda i: (i, 0)),
            pl.BlockSpec((BE, 3), lambda i: (i, 0)),
            pl.BlockSpec((128, 16), full),
            pl.BlockSpec((272, 512), full),
            pl.BlockSpec((1, 512), full),
            pl.BlockSpec((1, 512), full),
            pl.BlockSpec((1, 512), full),
            pl.BlockSpec((512, 144), full),
            pl.BlockSpec((1, 144), full),
        ],
        out_specs=pl.BlockSpec((64, BE), lambda i: (0, i)),
        out_shape=jax.ShapeDtypeStruct((64, e), jnp.float32),
    )(G1, G2, ef, rx, gsel, W1, b1, g1w, be1, W2, b2)


def _scatter_call(PT, dst):
    mesh = plsc.VectorSubcoreMesh(core_axis_name="c", subcore_axis_name="s")
    eph = N_EDGES // NC // 2
    nchunk = eph // SCH

    @functools.partial(
        pl.kernel,
        mesh=mesh,
        out_type=jax.ShapeDtypeStruct((NC, 2, 64, N_NODES), jnp.float32),
        compiler_params=pltpu.CompilerParams(needs_layout_passes=False),
        scratch_types=[
            pltpu.VMEM((2, SCH), jnp.int32),
            pltpu.VMEM((2, 8, SCH), jnp.float32),
        ] + [pltpu.VMEM((N_NODES,), jnp.float32) for _ in range(8)]
          + [pltpu.SemaphoreType.DMA] * 4,
    )
    def scatter_k(PT_hbm, dst_hbm, out_hbm, dstv, pbuf, *rest):
        accs = rest[:8]
        isem = rest[8:10]
        psem = rest[10:12]
        cid = lax.axis_index("c")
        sid = lax.axis_index("s")
        g = sid % 8
        hh = sid // 8
        ebase = (cid * 2 + hh) * eph

        zero16 = jnp.zeros((16,), jnp.float32)

        def zrow(r, carry):
            for c in range(8):
                accs[c][pl.ds(r * 16, 16)] = zero16
            return carry

        lax.fori_loop(0, N_NODES // 16, zrow, 0)

        def load(t, s):
            base = ebase + t * SCH
            pltpu.async_copy(dst_hbm.at[pl.ds(base, SCH)], dstv.at[s], isem[s])
            pltpu.async_copy(PT_hbm.at[pl.ds(8 * g, 8), pl.ds(base, SCH)],
                             pbuf.at[s], psem[s])

        def waitld(s):
            pltpu.make_async_copy(dst_hbm.at[pl.ds(0, SCH)], dstv.at[s],
                                  isem[s]).wait()
            pltpu.make_async_copy(PT_hbm.at[pl.ds(0, 8), pl.ds(0, SCH)],
                                  pbuf.at[s], psem[s]).wait()

        def compute(s):
            def group(j, carry2):
                rows = dstv[s, pl.ds(j * 16, 16)]
                for c in range(8):
                    vals = pbuf[s, c, pl.ds(j * 16, 16)]
                    plsc.addupdate_scatter(accs[c], [rows], vals)
                return carry2

            lax.fori_loop(0, SCH // 16, group, 0)

        load(0, 0)

        def body(u, carry):
            t0 = 2 * u

            @pl.when(t0 + 1 < nchunk)
            def _():
                load(t0 + 1, 1)

            waitld(0)
            compute(0)

            @pl.when(t0 + 1 < nchunk)
            def _():
                @pl.when(t0 + 2 < nchunk)
                def _():
                    load(t0 + 2, 0)

                waitld(1)
                compute(1)

            return carry

        lax.fori_loop(0, (nchunk + 1) // 2, body, 0)
        for c in range(8):
            pltpu.sync_copy(accs[c], out_hbm.at[cid, hh, 8 * g + c])

    return scatter_k(PT, dst)


def _fin_body(a_ref, b_ref, c_ref, d_ref, o_ref):
    acc = a_ref[...] + b_ref[...] + c_ref[...] + d_ref[...]
    s = acc[:16, :] + 1e-16
    rows = [jnp.sum(acc[16 + 16 * c: 32 + 16 * c, :] / s, axis=0, keepdims=True)
            * (1.0 / 16.0) for c in range(3)]
    o_ref[...] = jnp.concatenate(rows, axis=0)


def _fin_call(parts):
    n = parts[0].shape[1]
    return pl.pallas_call(
        _fin_body,
        grid=(1,),
        in_specs=[pl.BlockSpec((64, n), lambda i: (0, 0)) for _ in range(4)],
        out_specs=pl.BlockSpec((3, n), lambda i: (0, 0)),
        out_shape=jax.ShapeDtypeStruct((3, n), jnp.float32),
    )(*parts)


def kernel(h, rel_x, edge_feat, edge_index,
           k_W1, k_b1, k_g1, k_be1, k_W2, k_b2,
           v_W1, v_b1, v_g1, v_be1, v_W2, v_b2,
           q_W1, q_b1, q_g1, q_be1, q_W2, q_b2):
    src = edge_index[0].astype(jnp.int32)
    dst = edge_index[1].astype(jnp.int32)
    row = lambda x: x.reshape(1, -1)

    D = _node_call(h, q_W1, row(q_b1), row(q_g1), row(q_be1), q_W2, row(q_b2))
    G1, G2 = _gather_call(D, h, dst, src)

    bf = jnp.bfloat16
    gsel = jnp.kron(jnp.eye(16, dtype=bf), jnp.ones((8, 1), dtype=bf))
    W1 = jnp.concatenate([k_W1, v_W1], axis=1).astype(bf)
    b1 = row(jnp.concatenate([k_b1, v_b1]))
    g1w = row(jnp.concatenate([k_g1, v_g1]))
    be1 = row(jnp.concatenate([k_be1, v_be1]))
    W2 = jnp.block([
        [k_W2, jnp.zeros((256, 16), jnp.float32)],
        [jnp.zeros((256, 128), jnp.float32), v_W2],
    ]).astype(bf)
    b2 = row(jnp.concatenate([k_b2, v_b2]))
    P = _edge_call(G1, G2, edge_feat, rel_x, gsel, W1, b1, g1w, be1, W2, b2)

    partials = _scatter_call(P, dst)
    parts = [partials[0, 0], partials[0, 1], partials[1, 0], partials[1, 1]]
    return _fin_call(parts).T
